# Initial kernel scaffold; baseline (speedup 1.0000x reference)
import functools

import jax
import jax.numpy as jnp
from jax import lax
from jax.experimental import pallas as pl
from jax.experimental.pallas import tpu as pltpu
from jax.experimental.pallas import tpu_sc as plsc

N = 10000
E = 320000
D = 128
H1 = 64
H2 = 32
LANES = 16

NC = 2
NS = 16
NW = NC * NS
EPW = E // NW
CHUNK = 80
NCHUNK = EPW // CHUNK
RPT = N // NS
ZROWS = 125

PWS = (2.0, 1.5, 3.0, 0.8)


def _sc_body(src_hbm, dst_hbm, w_hbm, x_hbm, out_hbm,
             src_v, dst_v, w_v, rows_v, zbuf, agg_sh, sem):
    c = lax.axis_index("c")
    s = lax.axis_index("s")
    wid = c * NS + s

    zeros16 = jnp.zeros((LANES,), jnp.float32)

    def _zero_row(r, carry):
        for k in range(D // LANES):
            zbuf[r, pl.ds(k * LANES, LANES)] = zeros16
        return carry

    lax.fori_loop(0, ZROWS, _zero_row, 0)
    for rep in range(RPT // ZROWS):
        pltpu.sync_copy(zbuf, agg_sh.at[pl.ds(s * RPT + rep * ZROWS, ZROWS)])
    plsc.subcore_barrier()

    pltpu.sync_copy(src_hbm.at[wid], src_v)
    pltpu.sync_copy(dst_hbm.at[wid], dst_v)
    pltpu.sync_copy(w_hbm.at[wid], w_v)

    def _chunk(j, carry):
        pltpu.async_copy(x_hbm.at[src_v.at[j]], rows_v, sem).wait()

        def _scale(e, c2):
            wspl = plsc.load_gather(
                w_v, [jnp.full((LANES,), j, jnp.int32),
                      jnp.full((LANES,), e, jnp.int32)])
            for k in range(D // LANES):
                sl = pl.ds(k * LANES, LANES)
                rows_v[e, sl] = rows_v[e, sl] * wspl
            return c2

        lax.fori_loop(0, CHUNK, _scale, 0)
        pltpu.sync_copy(rows_v, agg_sh.at[dst_v.at[j]], add=True)
        return carry

    lax.fori_loop(0, NCHUNK, _chunk, 0)
    plsc.subcore_barrier()

    pltpu.sync_copy(agg_sh.at[pl.ds(s * RPT, RPT)],
                    out_hbm.at[c, pl.ds(s * RPT, RPT)])


_sc_aggregate = pl.kernel(
    _sc_body,
    out_type=jax.ShapeDtypeStruct((NC, N, D), jnp.float32),
    mesh=plsc.VectorSubcoreMesh(core_axis_name="c", subcore_axis_name="s"),
    scratch_types=[
        pltpu.VMEM((NCHUNK, CHUNK), jnp.int32),
        pltpu.VMEM((NCHUNK, CHUNK), jnp.int32),
        pltpu.VMEM((NCHUNK, CHUNK), jnp.float32),
        pltpu.VMEM((CHUNK, D), jnp.float32),
        pltpu.VMEM((ZROWS, D), jnp.float32),
        pltpu.VMEM_SHARED((N, D), jnp.float32),
        pltpu.SemaphoreType.DMA,
    ],
)


def _tc_body(agg2, Wg, bg, W1s, b1s, W2s, b2s, W3s, b3s, y, m, out_ref):
    agg = agg2[0] + agg2[1]
    emb = jnp.maximum(
        jnp.dot(agg, Wg[...], preferred_element_type=jnp.float32,
                precision=lax.Precision.HIGHEST) + bg[...], 0.0)
    yv = y[...]
    mv = m[...]
    denom = jnp.maximum(jnp.sum(mv), 1.0)
    for i in range(len(PWS)):
        h = jnp.maximum(
            jnp.dot(emb, W1s[i], preferred_element_type=jnp.float32,
                    precision=lax.Precision.HIGHEST) + b1s[i], 0.0)
        h = jnp.maximum(
            jnp.dot(h, W2s[i], preferred_element_type=jnp.float32,
                    precision=lax.Precision.HIGHEST) + b2s[i], 0.0)
        z = jnp.dot(h, W3s[i], preferred_element_type=jnp.float32,
                    precision=lax.Precision.HIGHEST) + b3s[i]
        pw = PWS[i]
        l = (1.0 - yv) * z + (1.0 + (pw - 1.0) * yv) * (
            jnp.log1p(jnp.exp(-jnp.abs(z))) + jnp.maximum(-z, 0.0))
        out_ref[i] = jnp.sum(l * mv) / denom


def _tc_dense(agg2, Wg, bg, W1s, b1s, W2s, b2s, W3s, b3s, y, m):
    return pl.pallas_call(
        _tc_body,
        out_shape=jax.ShapeDtypeStruct((len(PWS),), jnp.float32),
        out_specs=pl.BlockSpec(memory_space=pltpu.SMEM),
    )(agg2, Wg, bg, W1s, b1s, W2s, b2s, W3s, b3s, y, m)


def kernel(x, edge_index, edge_weight, y, mask, W_gnn, b_gnn,
           pr_W1, pr_b1, pr_W2, pr_b2, pr_W3, pr_b3,
           aux_W1, aux_b1, aux_W2, aux_b2, aux_W3, aux_b3):
    src3 = edge_index[0].reshape(NW, NCHUNK, CHUNK)
    dst3 = edge_index[1].reshape(NW, NCHUNK, CHUNK)
    w3 = edge_weight.reshape(NW, NCHUNK, CHUNK)
    agg2 = _sc_aggregate(src3, dst3, w3, x)

    W1s = jnp.concatenate([pr_W1[None], aux_W1])
    b1s = jnp.concatenate([pr_b1[None], aux_b1]).reshape(4, 1, H1)
    W2s = jnp.concatenate([pr_W2[None], aux_W2])
    b2s = jnp.concatenate([pr_b2[None], aux_b2]).reshape(4, 1, H2)
    W3s = jnp.concatenate([pr_W3[None], aux_W3])
    b3s = jnp.concatenate([pr_b3[None], aux_b3]).reshape(4, 1, 1)

    return _tc_dense(agg2, W_gnn, b_gnn.reshape(1, D), W1s, b1s, W2s, b2s,
                     W3s, b3s, y.reshape(N, 1), mask.astype(jnp.float32).reshape(N, 1))


# speedup vs baseline: 3.1715x; 3.1715x over previous
import jax
import jax.numpy as jnp
from jax import lax
from jax.experimental import pallas as pl
from jax.experimental.pallas import tpu as pltpu
from jax.experimental.pallas import tpu_sc as plsc

N = 10000
E = 320000
D = 128
H1 = 64
H2 = 32
LANES = 16

NC = 2
NS = 16
DH = D // NC
EPT = E // NS
CHUNK = 80
NCHUNK = EPT // CHUNK
RPT = 624
ZROWS = 208

PWS = (2.0, 1.5, 3.0, 0.8)


def _sc_body(src_hbm, dst_hbm, w_hbm, x_hbm, out_hbm,
             src_v, dst_v, w_v, rows_v, zbuf, agg_sh, sem):
    c = lax.axis_index("c")
    s = lax.axis_index("s")

    zeros16 = jnp.zeros((LANES,), jnp.float32)

    def _zero_row(r, carry):
        for k in range(DH // LANES):
            zbuf[r, pl.ds(k * LANES, LANES)] = zeros16
        return carry

    lax.fori_loop(0, ZROWS, _zero_row, 0)
    base = pl.multiple_of(s * RPT, 16)
    for rep in range(RPT // ZROWS):
        off = pl.multiple_of(base + rep * ZROWS, 16)
        pltpu.sync_copy(zbuf, agg_sh.at[pl.ds(off, ZROWS)])

    @pl.when(s == NS - 1)
    def _zero_tail():
        pltpu.sync_copy(zbuf.at[pl.ds(0, 16)], agg_sh.at[pl.ds(NS * RPT, 16)])

    plsc.subcore_barrier()

    pltpu.sync_copy(src_hbm.at[s], src_v)
    pltpu.sync_copy(dst_hbm.at[s], dst_v)
    pltpu.sync_copy(w_hbm.at[s], w_v)

    def _chunk(j, carry):
        pltpu.async_copy(x_hbm.at[c].at[src_v.at[j]], rows_v, sem).wait()

        def _scale(e, c2):
            wwin = w_v[pl.ds(j * CHUNK + e, LANES)]
            wspl = jnp.full((LANES,), wwin[0])
            for k in range(DH // LANES):
                sl = pl.ds(k * LANES, LANES)
                rows_v[e, sl] = rows_v[e, sl] * wspl
            return c2

        lax.fori_loop(0, CHUNK, _scale, 0)
        pltpu.sync_copy(rows_v, agg_sh.at[dst_v.at[j]], add=True)
        return carry

    lax.fori_loop(0, NCHUNK, _chunk, 0)
    plsc.subcore_barrier()

    for rep in range(RPT // ZROWS):
        off = pl.multiple_of(base + rep * ZROWS, 16)
        pltpu.sync_copy(agg_sh.at[pl.ds(off, ZROWS)],
                        out_hbm.at[c, pl.ds(off, ZROWS)])

    @pl.when(s == NS - 1)
    def _write_tail():
        pltpu.sync_copy(agg_sh.at[pl.ds(NS * RPT, 16)],
                        out_hbm.at[c, pl.ds(NS * RPT, 16)])


_sc_aggregate = pl.kernel(
    _sc_body,
    out_type=jax.ShapeDtypeStruct((NC, N, DH), jnp.float32),
    mesh=plsc.VectorSubcoreMesh(core_axis_name="c", subcore_axis_name="s"),
    scratch_types=[
        pltpu.VMEM((NCHUNK, CHUNK), jnp.int32),
        pltpu.VMEM((NCHUNK, CHUNK), jnp.int32),
        pltpu.VMEM((EPT + LANES,), jnp.float32),
        pltpu.VMEM((CHUNK, DH), jnp.float32),
        pltpu.VMEM((ZROWS, DH), jnp.float32),
        pltpu.VMEM_SHARED((N, DH), jnp.float32),
        pltpu.SemaphoreType.DMA,
    ],
    compiler_params=pltpu.CompilerParams(use_tc_tiling_on_sc=False),
)


NB = 2000
NBLK = N // NB


def _tc_body(agg2, Wg, bg, W1s, b1s, W2s, b2s, W3s, b3s, y, m, out_ref, acc):
    i = pl.program_id(0)

    @pl.when(i == 0)
    def _init():
        for k in range(len(PWS) + 1):
            acc[k] = 0.0

    emb = jnp.maximum(
        jnp.dot(agg2[0], Wg[:DH], preferred_element_type=jnp.float32,
                precision=lax.Precision.HIGHEST)
        + jnp.dot(agg2[1], Wg[DH:], preferred_element_type=jnp.float32,
                  precision=lax.Precision.HIGHEST)
        + bg[...], 0.0)
    yv = y[...]
    mv = m[...]
    acc[len(PWS)] = acc[len(PWS)] + jnp.sum(mv)
    for hd in range(len(PWS)):
        h = jnp.maximum(
            jnp.dot(emb, W1s[hd], preferred_element_type=jnp.float32,
                    precision=lax.Precision.HIGHEST) + b1s[hd], 0.0)
        h = jnp.maximum(
            jnp.dot(h, W2s[hd], preferred_element_type=jnp.float32,
                    precision=lax.Precision.HIGHEST) + b2s[hd], 0.0)
        z = jnp.dot(h, W3s[hd], preferred_element_type=jnp.float32,
                    precision=lax.Precision.HIGHEST) + b3s[hd]
        pw = PWS[hd]
        l = (1.0 - yv) * z + (1.0 + (pw - 1.0) * yv) * (
            jnp.log1p(jnp.exp(-jnp.abs(z))) + jnp.maximum(-z, 0.0))
        acc[hd] = acc[hd] + jnp.sum(l * mv)

    @pl.when(i == NBLK - 1)
    def _fin():
        denom = jnp.maximum(acc[len(PWS)], 1.0)
        for hd in range(len(PWS)):
            out_ref[hd] = acc[hd] / denom


def _tc_dense(agg2, Wg, bg, W1s, b1s, W2s, b2s, W3s, b3s, y, m):
    full = lambda shape: pl.BlockSpec(shape, lambda i: tuple(0 for _ in shape))
    return pl.pallas_call(
        _tc_body,
        grid=(NBLK,),
        in_specs=[
            pl.BlockSpec((NC, NB, DH), lambda i: (0, i, 0)),
            full((D, D)),
            full((1, D)),
            full((4, D, H1)),
            full((4, 1, H1)),
            full((4, H1, H2)),
            full((4, 1, H2)),
            full((4, H2, 1)),
            full((4, 1, 1)),
            pl.BlockSpec((NB, 1), lambda i: (i, 0)),
            pl.BlockSpec((NB, 1), lambda i: (i, 0)),
        ],
        out_shape=jax.ShapeDtypeStruct((len(PWS),), jnp.float32),
        out_specs=pl.BlockSpec(memory_space=pltpu.SMEM),
        scratch_shapes=[pltpu.SMEM((len(PWS) + 1,), jnp.float32)],
        compiler_params=pltpu.CompilerParams(
            dimension_semantics=("arbitrary",)),
    )(agg2, Wg, bg, W1s, b1s, W2s, b2s, W3s, b3s, y, m)


def kernel(x, edge_index, edge_weight, y, mask, W_gnn, b_gnn,
           pr_W1, pr_b1, pr_W2, pr_b2, pr_W3, pr_b3,
           aux_W1, aux_b1, aux_W2, aux_b2, aux_W3, aux_b3):
    src3 = edge_index[0].reshape(NS, NCHUNK, CHUNK)
    dst3 = edge_index[1].reshape(NS, NCHUNK, CHUNK)
    w2 = jnp.pad(edge_weight.reshape(NS, EPT), ((0, 0), (0, LANES)))
    x2 = x.reshape(N, NC, DH).swapaxes(0, 1)
    agg2 = _sc_aggregate(src3, dst3, w2, x2)

    W1s = jnp.concatenate([pr_W1[None], aux_W1])
    b1s = jnp.concatenate([pr_b1[None], aux_b1]).reshape(4, 1, H1)
    W2s = jnp.concatenate([pr_W2[None], aux_W2])
    b2s = jnp.concatenate([pr_b2[None], aux_b2]).reshape(4, 1, H2)
    W3s = jnp.concatenate([pr_W3[None], aux_W3])
    b3s = jnp.concatenate([pr_b3[None], aux_b3]).reshape(4, 1, 1)

    return _tc_dense(agg2, W_gnn, b_gnn.reshape(1, D), W1s, b1s, W2s, b2s,
                     W3s, b3s, y.reshape(N, 1), mask.astype(jnp.float32).reshape(N, 1))


# speedup vs baseline: 5.9241x; 1.8679x over previous
import jax
import jax.numpy as jnp
from jax import lax
from jax.experimental import pallas as pl
from jax.experimental.pallas import tpu as pltpu
from jax.experimental.pallas import tpu_sc as plsc

N = 10000
E = 320000
D = 128
H1 = 64
H2 = 32
LANES = 16

NC = 2
NS = 16
DH = D // NC
EPT = E // NS
CHUNK = 40
NCHUNK = EPT // CHUNK
NBUF = 4
RPT = 624
ZROWS = 208

PWS = (2.0, 1.5, 3.0, 0.8)


def _sc_body(src_hbm, dst_hbm, w_hbm, x_hbm, out_hbm,
             src_v, dst_v, w_v, rows0, rows1, rows2, rows3, zbuf, agg_sh,
             sg0, sg1, sg2, sg3, ss0, ss1, ss2, ss3):
    rows = (rows0, rows1, rows2, rows3)
    sg = (sg0, sg1, sg2, sg3)
    ss = (ss0, ss1, ss2, ss3)
    c = lax.axis_index("c")
    s = lax.axis_index("s")

    zeros16 = jnp.zeros((LANES,), jnp.float32)

    def _zero_row(r, carry):
        for k in range(DH // LANES):
            zbuf[r, pl.ds(k * LANES, LANES)] = zeros16
        return carry

    lax.fori_loop(0, ZROWS, _zero_row, 0)
    base = pl.multiple_of(s * RPT, 16)
    for rep in range(RPT // ZROWS):
        off = pl.multiple_of(base + rep * ZROWS, 16)
        pltpu.sync_copy(zbuf, agg_sh.at[pl.ds(off, ZROWS)])

    @pl.when(s == NS - 1)
    def _zero_tail():
        pltpu.sync_copy(zbuf.at[pl.ds(0, 16)], agg_sh.at[pl.ds(NS * RPT, 16)])

    plsc.subcore_barrier()

    pltpu.sync_copy(src_hbm.at[s], src_v)
    pltpu.sync_copy(dst_hbm.at[s], dst_v)
    pltpu.sync_copy(w_hbm.at[s], w_v)

    def _gather(jj, b):
        pltpu.async_copy(x_hbm.at[c].at[src_v.at[jj]], rows[b], sg[b])

    for b in range(NBUF - 1):
        _gather(jnp.int32(b), b)

    def _quad(q, carry):
        for b in range(NBUF):
            j = q * NBUF + b
            pltpu.make_async_copy(x_hbm.at[c].at[src_v.at[j]],
                                  rows[b], sg[b]).wait()
            for g in range(CHUNK // 8):
                w16 = w_v[pl.ds(j * CHUNK + g * 8, LANES)]
                for i in range(8):
                    wspl = jnp.full((LANES,), w16[i])
                    e = g * 8 + i
                    for k in range(DH // LANES):
                        sl = pl.ds(k * LANES, LANES)
                        rows[b][e, sl] = rows[b][e, sl] * wspl
            pltpu.async_copy(rows[b], agg_sh.at[dst_v.at[j]], ss[b], add=True)
            pb = (b + NBUF - 1) % NBUF

            def _scat_wait():
                pltpu.make_async_copy(rows[pb],
                                      agg_sh.at[dst_v.at[j]], ss[pb]).wait()

            if b == 0:
                @pl.when(q > 0)
                def _scat_wait0():
                    _scat_wait()
            else:
                _scat_wait()

            @pl.when(j + NBUF - 1 < NCHUNK)
            def _refill():
                _gather(j + NBUF - 1, pb)
        return carry

    lax.fori_loop(0, NCHUNK // NBUF, _quad, 0)
    pltpu.make_async_copy(rows[NBUF - 1], agg_sh.at[dst_v.at[0]],
                          ss[NBUF - 1]).wait()
    plsc.subcore_barrier()

    for rep in range(RPT // ZROWS):
        off = pl.multiple_of(base + rep * ZROWS, 16)
        pltpu.sync_copy(agg_sh.at[pl.ds(off, ZROWS)],
                        out_hbm.at[c, pl.ds(off, ZROWS)])

    @pl.when(s == NS - 1)
    def _write_tail():
        pltpu.sync_copy(agg_sh.at[pl.ds(NS * RPT, 16)],
                        out_hbm.at[c, pl.ds(NS * RPT, 16)])


_sc_aggregate = pl.kernel(
    _sc_body,
    out_type=jax.ShapeDtypeStruct((NC, N, DH), jnp.float32),
    mesh=plsc.VectorSubcoreMesh(core_axis_name="c", subcore_axis_name="s"),
    scratch_types=[
        pltpu.VMEM((NCHUNK, CHUNK), jnp.int32),
        pltpu.VMEM((NCHUNK, CHUNK), jnp.int32),
        pltpu.VMEM((EPT + LANES,), jnp.float32),
        pltpu.VMEM((CHUNK, DH), jnp.float32),
        pltpu.VMEM((CHUNK, DH), jnp.float32),
        pltpu.VMEM((CHUNK, DH), jnp.float32),
        pltpu.VMEM((CHUNK, DH), jnp.float32),
        pltpu.VMEM((ZROWS, DH), jnp.float32),
        pltpu.VMEM_SHARED((N, DH), jnp.float32),
        pltpu.SemaphoreType.DMA,
        pltpu.SemaphoreType.DMA,
        pltpu.SemaphoreType.DMA,
        pltpu.SemaphoreType.DMA,
        pltpu.SemaphoreType.DMA,
        pltpu.SemaphoreType.DMA,
        pltpu.SemaphoreType.DMA,
        pltpu.SemaphoreType.DMA,
    ],
    compiler_params=pltpu.CompilerParams(use_tc_tiling_on_sc=False),
)


NB = 2000
NBLK = N // NB


def _tc_body(agg2, Wg, bg, W1s, b1s, W2s, b2s, W3s, b3s, y, m, out_ref, acc):
    i = pl.program_id(0)

    @pl.when(i == 0)
    def _init():
        for k in range(len(PWS) + 1):
            acc[k] = 0.0

    emb = jnp.maximum(
        jnp.dot(agg2[0], Wg[:DH], preferred_element_type=jnp.float32,
                precision=lax.Precision.HIGHEST)
        + jnp.dot(agg2[1], Wg[DH:], preferred_element_type=jnp.float32,
                  precision=lax.Precision.HIGHEST)
        + bg[...], 0.0)
    yv = y[...]
    mv = m[...]
    acc[len(PWS)] = acc[len(PWS)] + jnp.sum(mv)
    for hd in range(len(PWS)):
        h = jnp.maximum(
            jnp.dot(emb, W1s[hd], preferred_element_type=jnp.float32,
                    precision=lax.Precision.HIGHEST) + b1s[hd], 0.0)
        h = jnp.maximum(
            jnp.dot(h, W2s[hd], preferred_element_type=jnp.float32,
                    precision=lax.Precision.HIGHEST) + b2s[hd], 0.0)
        z = jnp.dot(h, W3s[hd], preferred_element_type=jnp.float32,
                    precision=lax.Precision.HIGHEST) + b3s[hd]
        pw = PWS[hd]
        l = (1.0 - yv) * z + (1.0 + (pw - 1.0) * yv) * (
            jnp.log1p(jnp.exp(-jnp.abs(z))) + jnp.maximum(-z, 0.0))
        acc[hd] = acc[hd] + jnp.sum(l * mv)

    @pl.when(i == NBLK - 1)
    def _fin():
        denom = jnp.maximum(acc[len(PWS)], 1.0)
        for hd in range(len(PWS)):
            out_ref[hd] = acc[hd] / denom


def _tc_dense(agg2, Wg, bg, W1s, b1s, W2s, b2s, W3s, b3s, y, m):
    full = lambda shape: pl.BlockSpec(shape, lambda i: tuple(0 for _ in shape))
    return pl.pallas_call(
        _tc_body,
        grid=(NBLK,),
        in_specs=[
            pl.BlockSpec((NC, NB, DH), lambda i: (0, i, 0)),
            full((D, D)),
            full((1, D)),
            full((4, D, H1)),
            full((4, 1, H1)),
            full((4, H1, H2)),
            full((4, 1, H2)),
            full((4, H2, 1)),
            full((4, 1, 1)),
            pl.BlockSpec((NB, 1), lambda i: (i, 0)),
            pl.BlockSpec((NB, 1), lambda i: (i, 0)),
        ],
        out_shape=jax.ShapeDtypeStruct((len(PWS),), jnp.float32),
        out_specs=pl.BlockSpec(memory_space=pltpu.SMEM),
        scratch_shapes=[pltpu.SMEM((len(PWS) + 1,), jnp.float32)],
        compiler_params=pltpu.CompilerParams(
            dimension_semantics=("arbitrary",)),
    )(agg2, Wg, bg, W1s, b1s, W2s, b2s, W3s, b3s, y, m)


def kernel(x, edge_index, edge_weight, y, mask, W_gnn, b_gnn,
           pr_W1, pr_b1, pr_W2, pr_b2, pr_W3, pr_b3,
           aux_W1, aux_b1, aux_W2, aux_b2, aux_W3, aux_b3):
    src3 = edge_index[0].reshape(NS, NCHUNK, CHUNK)
    dst3 = edge_index[1].reshape(NS, NCHUNK, CHUNK)
    w2 = jnp.pad(edge_weight.reshape(NS, EPT), ((0, 0), (0, LANES)))
    x2 = x.reshape(N, NC, DH).swapaxes(0, 1)
    agg2 = _sc_aggregate(src3, dst3, w2, x2)

    W1s = jnp.concatenate([pr_W1[None], aux_W1])
    b1s = jnp.concatenate([pr_b1[None], aux_b1]).reshape(4, 1, H1)
    W2s = jnp.concatenate([pr_W2[None], aux_W2])
    b2s = jnp.concatenate([pr_b2[None], aux_b2]).reshape(4, 1, H2)
    W3s = jnp.concatenate([pr_W3[None], aux_W3])
    b3s = jnp.concatenate([pr_b3[None], aux_b3]).reshape(4, 1, 1)

    return _tc_dense(agg2, W_gnn, b_gnn.reshape(1, D), W1s, b1s, W2s, b2s,
                     W3s, b3s, y.reshape(N, 1), mask.astype(jnp.float32).reshape(N, 1))


# speedup vs baseline: 7.7125x; 1.3019x over previous
import jax
import jax.numpy as jnp
from jax import lax
from jax.experimental import pallas as pl
from jax.experimental.pallas import tpu as pltpu
from jax.experimental.pallas import tpu_sc as plsc

N = 10000
E = 320000
D = 128
H1 = 64
H2 = 32
LANES = 16

NC = 2
NS = 16
DH = D // NC
EPT = E // NS
CHUNK = 40
NCHUNK = EPT // CHUNK
NBUF = 4
RPT = 624
ZROWS = 208

PWS = (2.0, 1.5, 3.0, 0.8)


def _sc_body(src_hbm, dst_hbm, w_hbm, x_hbm, out_hbm,
             src_v, dst_v, w_v, rows0, rows1, rows2, rows3, zbuf, agg_sh,
             sg0, sg1, sg2, sg3, ss0, ss1, ss2, ss3):
    rows = (rows0, rows1, rows2, rows3)
    sg = (sg0, sg1, sg2, sg3)
    ss = (ss0, ss1, ss2, ss3)
    c = lax.axis_index("c")
    s = lax.axis_index("s")

    zeros16 = jnp.zeros((LANES,), jnp.float32)

    def _zero_row(r, carry):
        for k in range(DH // LANES):
            zbuf[r, pl.ds(k * LANES, LANES)] = zeros16
        return carry

    lax.fori_loop(0, ZROWS, _zero_row, 0)
    base = pl.multiple_of(s * RPT, 16)
    for rep in range(RPT // ZROWS):
        off = pl.multiple_of(base + rep * ZROWS, 16)
        pltpu.sync_copy(zbuf, agg_sh.at[pl.ds(off, ZROWS)])

    @pl.when(s == NS - 1)
    def _zero_tail():
        pltpu.sync_copy(zbuf.at[pl.ds(0, 16)], agg_sh.at[pl.ds(NS * RPT, 16)])

    plsc.subcore_barrier()

    pltpu.sync_copy(src_hbm.at[c].at[s], src_v)
    pltpu.sync_copy(dst_hbm.at[s], dst_v)
    pltpu.sync_copy(w_hbm.at[s], w_v)

    def _gather(jj, b):
        pltpu.async_copy(x_hbm.at[src_v.at[jj]], rows[b], sg[b])

    for b in range(NBUF - 1):
        _gather(jnp.int32(b), b)

    def _quad(q, carry):
        for b in range(NBUF):
            j = q * NBUF + b
            pltpu.make_async_copy(x_hbm.at[src_v.at[j]],
                                  rows[b], sg[b]).wait()
            for g in range(CHUNK // 8):
                w16 = w_v[pl.ds(j * CHUNK + g * 8, LANES)]
                for i in range(8):
                    wspl = jnp.full((LANES,), w16[i])
                    e = g * 8 + i
                    for k in range(DH // LANES):
                        sl = pl.ds(k * LANES, LANES)
                        rows[b][e, sl] = rows[b][e, sl] * wspl
            pltpu.async_copy(rows[b], agg_sh.at[dst_v.at[j]], ss[b], add=True)
            pb = (b + NBUF - 1) % NBUF

            def _scat_wait():
                pltpu.make_async_copy(rows[pb],
                                      agg_sh.at[dst_v.at[j]], ss[pb]).wait()

            if b == 0:
                @pl.when(q > 0)
                def _scat_wait0():
                    _scat_wait()
            else:
                _scat_wait()

            @pl.when(j + NBUF - 1 < NCHUNK)
            def _refill():
                _gather(j + NBUF - 1, pb)
        return carry

    lax.fori_loop(0, NCHUNK // NBUF, _quad, 0)
    pltpu.make_async_copy(rows[NBUF - 1], agg_sh.at[dst_v.at[0]],
                          ss[NBUF - 1]).wait()
    plsc.subcore_barrier()

    for rep in range(RPT // ZROWS):
        off = pl.multiple_of(base + rep * ZROWS, 16)
        pltpu.sync_copy(agg_sh.at[pl.ds(off, ZROWS)],
                        out_hbm.at[c, pl.ds(off, ZROWS)])

    @pl.when(s == NS - 1)
    def _write_tail():
        pltpu.sync_copy(agg_sh.at[pl.ds(NS * RPT, 16)],
                        out_hbm.at[c, pl.ds(NS * RPT, 16)])


_sc_aggregate = pl.kernel(
    _sc_body,
    out_type=jax.ShapeDtypeStruct((NC, N, DH), jnp.float32),
    mesh=plsc.VectorSubcoreMesh(core_axis_name="c", subcore_axis_name="s"),
    scratch_types=[
        pltpu.VMEM((NCHUNK, CHUNK), jnp.int32),
        pltpu.VMEM((NCHUNK, CHUNK), jnp.int32),
        pltpu.VMEM((EPT + LANES,), jnp.float32),
        pltpu.VMEM((CHUNK, DH), jnp.float32),
        pltpu.VMEM((CHUNK, DH), jnp.float32),
        pltpu.VMEM((CHUNK, DH), jnp.float32),
        pltpu.VMEM((CHUNK, DH), jnp.float32),
        pltpu.VMEM((ZROWS, DH), jnp.float32),
        pltpu.VMEM_SHARED((N, DH), jnp.float32),
        pltpu.SemaphoreType.DMA,
        pltpu.SemaphoreType.DMA,
        pltpu.SemaphoreType.DMA,
        pltpu.SemaphoreType.DMA,
        pltpu.SemaphoreType.DMA,
        pltpu.SemaphoreType.DMA,
        pltpu.SemaphoreType.DMA,
        pltpu.SemaphoreType.DMA,
    ],
    compiler_params=pltpu.CompilerParams(use_tc_tiling_on_sc=False),
)


NB = 2000
NBLK = N // NB


def _tc_body(agg2, Wg, bg, W1s, b1s, W2s, b2s, W3s, b3s, y, m, out_ref, acc):
    i = pl.program_id(0)

    @pl.when(i == 0)
    def _init():
        for k in range(len(PWS) + 1):
            acc[k] = 0.0

    emb = jnp.maximum(
        jnp.dot(agg2[0], Wg[:DH], preferred_element_type=jnp.float32)
        + jnp.dot(agg2[1], Wg[DH:], preferred_element_type=jnp.float32)
        + bg[...], 0.0)
    yv = y[...]
    mv = m[...]
    acc[len(PWS)] = acc[len(PWS)] + jnp.sum(mv)
    for hd in range(len(PWS)):
        h = jnp.maximum(
            jnp.dot(emb, W1s[hd], preferred_element_type=jnp.float32)
            + b1s[hd], 0.0)
        h = jnp.maximum(
            jnp.dot(h, W2s[hd], preferred_element_type=jnp.float32)
            + b2s[hd], 0.0)
        z = jnp.dot(h, W3s[hd], preferred_element_type=jnp.float32) + b3s[hd]
        pw = PWS[hd]
        l = (1.0 - yv) * z + (1.0 + (pw - 1.0) * yv) * (
            jnp.log1p(jnp.exp(-jnp.abs(z))) + jnp.maximum(-z, 0.0))
        acc[hd] = acc[hd] + jnp.sum(l * mv)

    @pl.when(i == NBLK - 1)
    def _fin():
        denom = jnp.maximum(acc[len(PWS)], 1.0)
        for hd in range(len(PWS)):
            out_ref[hd] = acc[hd] / denom


def _tc_dense(agg2, Wg, bg, W1s, b1s, W2s, b2s, W3s, b3s, y, m):
    full = lambda shape: pl.BlockSpec(shape, lambda i: tuple(0 for _ in shape))
    return pl.pallas_call(
        _tc_body,
        grid=(NBLK,),
        in_specs=[
            pl.BlockSpec((NC, NB, DH), lambda i: (0, i, 0)),
            full((D, D)),
            full((1, D)),
            full((4, D, H1)),
            full((4, 1, H1)),
            full((4, H1, H2)),
            full((4, 1, H2)),
            full((4, H2, 1)),
            full((4, 1, 1)),
            pl.BlockSpec((NB, 1), lambda i: (i, 0)),
            pl.BlockSpec((NB, 1), lambda i: (i, 0)),
        ],
        out_shape=jax.ShapeDtypeStruct((len(PWS),), jnp.float32),
        out_specs=pl.BlockSpec(memory_space=pltpu.SMEM),
        scratch_shapes=[pltpu.SMEM((len(PWS) + 1,), jnp.float32)],
        compiler_params=pltpu.CompilerParams(
            dimension_semantics=("arbitrary",)),
    )(agg2, Wg, bg, W1s, b1s, W2s, b2s, W3s, b3s, y, m)


def kernel(x, edge_index, edge_weight, y, mask, W_gnn, b_gnn,
           pr_W1, pr_b1, pr_W2, pr_b2, pr_W3, pr_b3,
           aux_W1, aux_b1, aux_W2, aux_b2, aux_W3, aux_b3):
    src3 = edge_index[0].reshape(NS, NCHUNK, CHUNK)
    src4 = 2 * src3[None] + jnp.arange(NC, dtype=jnp.int32)[:, None, None, None]
    dst3 = edge_index[1].reshape(NS, NCHUNK, CHUNK)
    w2 = jnp.pad(edge_weight.reshape(NS, EPT), ((0, 0), (0, LANES)))
    agg2 = _sc_aggregate(src4, dst3, w2, x.reshape(NC * N, DH))

    W1s = jnp.concatenate([pr_W1[None], aux_W1])
    b1s = jnp.concatenate([pr_b1[None], aux_b1]).reshape(4, 1, H1)
    W2s = jnp.concatenate([pr_W2[None], aux_W2])
    b2s = jnp.concatenate([pr_b2[None], aux_b2]).reshape(4, 1, H2)
    W3s = jnp.concatenate([pr_W3[None], aux_W3])
    b3s = jnp.concatenate([pr_b3[None], aux_b3]).reshape(4, 1, 1)

    return _tc_dense(agg2, W_gnn, b_gnn.reshape(1, D), W1s, b1s, W2s, b2s,
                     W3s, b3s, y.reshape(N, 1), mask.astype(jnp.float32).reshape(N, 1))


# speedup vs baseline: 9.2735x; 1.2024x over previous
import jax
import jax.numpy as jnp
from jax import lax
from jax.experimental import pallas as pl
from jax.experimental.pallas import tpu as pltpu
from jax.experimental.pallas import tpu_sc as plsc

N = 10000
E = 320000
D = 128
H1 = 64
H2 = 32
LANES = 16

NC = 2
NS = 16
DH = D // NC
EPT = E // NS
CHUNK = 32
NCHUNK = EPT // CHUNK
NBUF = 5
RPT = 624
ZROWS = 208

PWS = (2.0, 1.5, 3.0, 0.8)


def _sc_body(ei_hbm, w_hbm, x_hbm, out_hbm,
             src_v, dst_v, w_v, rows0, rows1, rows2, rows3, rows4,
             zbuf, agg_sh,
             sg0, sg1, sg2, sg3, sg4, ss0, ss1, ss2, ss3, ss4):
    rows = (rows0, rows1, rows2, rows3, rows4)
    sg = (sg0, sg1, sg2, sg3, sg4)
    ss = (ss0, ss1, ss2, ss3, ss4)
    c = lax.axis_index("c")
    s = lax.axis_index("s")

    zeros16 = jnp.zeros((LANES,), jnp.float32)

    def _zero_row(r, carry):
        for k in range(DH // LANES):
            zbuf[r, pl.ds(k * LANES, LANES)] = zeros16
        return carry

    lax.fori_loop(0, ZROWS, _zero_row, 0)
    base = pl.multiple_of(s * RPT, 16)
    for rep in range(RPT // ZROWS):
        off = pl.multiple_of(base + rep * ZROWS, 16)
        pltpu.sync_copy(zbuf, agg_sh.at[pl.ds(off, ZROWS)])

    @pl.when(s == NS - 1)
    def _zero_tail():
        pltpu.sync_copy(zbuf.at[pl.ds(0, 16)], agg_sh.at[pl.ds(NS * RPT, 16)])

    plsc.subcore_barrier()

    pltpu.sync_copy(ei_hbm.at[0].at[s], src_v)
    pltpu.sync_copy(ei_hbm.at[1].at[s], dst_v)
    pltpu.sync_copy(w_hbm.at[s], w_v)

    cvec = jnp.full((LANES,), c, jnp.int32)

    def _xform(i, carry):
        sl = pl.ds(i * LANES, LANES)
        v = src_v[sl]
        src_v[sl] = v + v + cvec
        return carry

    lax.fori_loop(0, EPT // LANES, _xform, 0)

    def _gather(jj, b):
        pltpu.async_copy(x_hbm.at[src_v.at[pl.ds(jj * CHUNK, CHUNK)]],
                         rows[b], sg[b])

    for b in range(NBUF - 1):
        _gather(jnp.int32(b), b)

    def _dstix(j):
        return dst_v.at[pl.ds(j * CHUNK, CHUNK)]

    def _round(q, carry):
        for b in range(NBUF):
            j = q * NBUF + b
            pltpu.make_async_copy(x_hbm.at[src_v.at[pl.ds(j * CHUNK, CHUNK)]],
                                  rows[b], sg[b]).wait()
            for g in range(CHUNK // LANES):
                w16 = w_v[pl.ds(j * CHUNK + g * LANES, LANES)]
                for i in range(LANES):
                    wspl = jnp.full((LANES,), w16[i])
                    e = g * LANES + i
                    for k in range(DH // LANES):
                        sl = pl.ds(k * LANES, LANES)
                        rows[b][e, sl] = rows[b][e, sl] * wspl
            pltpu.async_copy(rows[b], agg_sh.at[_dstix(j)], ss[b], add=True)
            pb = (b + NBUF - 1) % NBUF

            def _scat_wait():
                pltpu.make_async_copy(rows[pb],
                                      agg_sh.at[_dstix(j)], ss[pb]).wait()

            if b == 0:
                @pl.when(q > 0)
                def _scat_wait0():
                    _scat_wait()
            else:
                _scat_wait()

            @pl.when(j + NBUF - 1 < NCHUNK)
            def _refill():
                _gather(j + NBUF - 1, pb)
        return carry

    lax.fori_loop(0, NCHUNK // NBUF, _round, 0)
    pltpu.make_async_copy(rows[NBUF - 1], agg_sh.at[_dstix(0)],
                          ss[NBUF - 1]).wait()
    plsc.subcore_barrier()

    for rep in range(RPT // ZROWS):
        off = pl.multiple_of(base + rep * ZROWS, 16)
        pltpu.sync_copy(agg_sh.at[pl.ds(off, ZROWS)],
                        out_hbm.at[c, pl.ds(off, ZROWS)])

    @pl.when(s == NS - 1)
    def _write_tail():
        pltpu.sync_copy(agg_sh.at[pl.ds(NS * RPT, 16)],
                        out_hbm.at[c, pl.ds(NS * RPT, 16)])


_sc_aggregate = pl.kernel(
    _sc_body,
    out_type=jax.ShapeDtypeStruct((NC, N, DH), jnp.float32),
    mesh=plsc.VectorSubcoreMesh(core_axis_name="c", subcore_axis_name="s"),
    scratch_types=[
        pltpu.VMEM((EPT,), jnp.int32),
        pltpu.VMEM((EPT,), jnp.int32),
        pltpu.VMEM((EPT,), jnp.float32),
        pltpu.VMEM((CHUNK, DH), jnp.float32),
        pltpu.VMEM((CHUNK, DH), jnp.float32),
        pltpu.VMEM((CHUNK, DH), jnp.float32),
        pltpu.VMEM((CHUNK, DH), jnp.float32),
        pltpu.VMEM((CHUNK, DH), jnp.float32),
        pltpu.VMEM((ZROWS, DH), jnp.float32),
        pltpu.VMEM_SHARED((N, DH), jnp.float32),
        pltpu.SemaphoreType.DMA,
        pltpu.SemaphoreType.DMA,
        pltpu.SemaphoreType.DMA,
        pltpu.SemaphoreType.DMA,
        pltpu.SemaphoreType.DMA,
        pltpu.SemaphoreType.DMA,
        pltpu.SemaphoreType.DMA,
        pltpu.SemaphoreType.DMA,
        pltpu.SemaphoreType.DMA,
        pltpu.SemaphoreType.DMA,
    ],
    compiler_params=pltpu.CompilerParams(use_tc_tiling_on_sc=False),
)


NB = 2000
NBLK = N // NB


def _tc_body(agg2, Wg, bg, W1s, b1s, W2s, b2s, W3s, b3s, y, m, out_ref, acc):
    i = pl.program_id(0)

    @pl.when(i == 0)
    def _init():
        for k in range(len(PWS) + 1):
            acc[k] = 0.0

    emb = jnp.maximum(
        jnp.dot(agg2[0], Wg[:DH], preferred_element_type=jnp.float32)
        + jnp.dot(agg2[1], Wg[DH:], preferred_element_type=jnp.float32)
        + bg[...], 0.0)
    yv = y[...]
    mv = m[...]
    acc[len(PWS)] = acc[len(PWS)] + jnp.sum(mv)
    for hd in range(len(PWS)):
        h = jnp.maximum(
            jnp.dot(emb, W1s[hd], preferred_element_type=jnp.float32)
            + b1s[hd], 0.0)
        h = jnp.maximum(
            jnp.dot(h, W2s[hd], preferred_element_type=jnp.float32)
            + b2s[hd], 0.0)
        z = jnp.dot(h, W3s[hd], preferred_element_type=jnp.float32) + b3s[hd]
        pw = PWS[hd]
        l = (1.0 - yv) * z + (1.0 + (pw - 1.0) * yv) * (
            jnp.log1p(jnp.exp(-jnp.abs(z))) + jnp.maximum(-z, 0.0))
        acc[hd] = acc[hd] + jnp.sum(l * mv)

    @pl.when(i == NBLK - 1)
    def _fin():
        denom = jnp.maximum(acc[len(PWS)], 1.0)
        for hd in range(len(PWS)):
            out_ref[hd] = acc[hd] / denom


def _tc_dense(agg2, Wg, bg, W1s, b1s, W2s, b2s, W3s, b3s, y, m):
    full = lambda shape: pl.BlockSpec(shape, lambda i: tuple(0 for _ in shape))
    return pl.pallas_call(
        _tc_body,
        grid=(NBLK,),
        in_specs=[
            pl.BlockSpec((NC, NB, DH), lambda i: (0, i, 0)),
            full((D, D)),
            full((1, D)),
            full((4, D, H1)),
            full((4, 1, H1)),
            full((4, H1, H2)),
            full((4, 1, H2)),
            full((4, H2, 1)),
            full((4, 1, 1)),
            pl.BlockSpec((NB, 1), lambda i: (i, 0)),
            pl.BlockSpec((NB, 1), lambda i: (i, 0)),
        ],
        out_shape=jax.ShapeDtypeStruct((len(PWS),), jnp.float32),
        out_specs=pl.BlockSpec(memory_space=pltpu.SMEM),
        scratch_shapes=[pltpu.SMEM((len(PWS) + 1,), jnp.float32)],
        compiler_params=pltpu.CompilerParams(
            dimension_semantics=("arbitrary",)),
    )(agg2, Wg, bg, W1s, b1s, W2s, b2s, W3s, b3s, y, m)


def kernel(x, edge_index, edge_weight, y, mask, W_gnn, b_gnn,
           pr_W1, pr_b1, pr_W2, pr_b2, pr_W3, pr_b3,
           aux_W1, aux_b1, aux_W2, aux_b2, aux_W3, aux_b3):
    agg2 = _sc_aggregate(edge_index.reshape(2, NS, EPT),
                         edge_weight.reshape(NS, EPT),
                         x.reshape(NC * N, DH))

    W1s = jnp.concatenate([pr_W1[None], aux_W1])
    b1s = jnp.concatenate([pr_b1[None], aux_b1]).reshape(4, 1, H1)
    W2s = jnp.concatenate([pr_W2[None], aux_W2])
    b2s = jnp.concatenate([pr_b2[None], aux_b2]).reshape(4, 1, H2)
    W3s = jnp.concatenate([pr_W3[None], aux_W3])
    b3s = jnp.concatenate([pr_b3[None], aux_b3]).reshape(4, 1, 1)

    return _tc_dense(agg2, W_gnn, b_gnn.reshape(1, D), W1s, b1s, W2s, b2s,
                     W3s, b3s, y.reshape(N, 1), mask.astype(jnp.float32).reshape(N, 1))


# speedup vs baseline: 9.9963x; 1.0779x over previous
import jax
import jax.numpy as jnp
from jax import lax
from jax.experimental import pallas as pl
from jax.experimental.pallas import tpu as pltpu
from jax.experimental.pallas import tpu_sc as plsc

N = 10000
E = 320000
D = 128
H1 = 64
H2 = 32
LANES = 16

NC = 2
NS = 16
DH = D // NC
EPT = E // NS
CHUNK = 32
NCHUNK = EPT // CHUNK
NBUF = 5
RPT = 624
ZROWS = 208

PWS = (2.0, 1.5, 3.0, 0.8)


def _sc_body(ei_hbm, w_hbm, x_hbm, out_hbm,
             src_v, dst_v, w_v, rows0, rows1, rows2, rows3, rows4,
             zbuf, agg_sh,
             sg0, sg1, sg2, sg3, sg4, ss0, ss1, ss2, ss3, ss4):
    rows = (rows0, rows1, rows2, rows3, rows4)
    sg = (sg0, sg1, sg2, sg3, sg4)
    ss = (ss0, ss1, ss2, ss3, ss4)
    c = lax.axis_index("c")
    s = lax.axis_index("s")

    zeros16 = jnp.zeros((LANES,), jnp.float32)

    def _zero_row(r, carry):
        for k in range(DH // LANES):
            zbuf[r, pl.ds(k * LANES, LANES)] = zeros16
        return carry

    lax.fori_loop(0, ZROWS, _zero_row, 0)
    base = pl.multiple_of(s * RPT, 16)
    for rep in range(RPT // ZROWS):
        off = pl.multiple_of(base + rep * ZROWS, 16)
        pltpu.sync_copy(zbuf, agg_sh.at[pl.ds(off, ZROWS)])

    @pl.when(s == NS - 1)
    def _zero_tail():
        pltpu.sync_copy(zbuf.at[pl.ds(0, 16)], agg_sh.at[pl.ds(NS * RPT, 16)])

    plsc.subcore_barrier()

    pltpu.sync_copy(ei_hbm.at[0].at[s], src_v)
    pltpu.sync_copy(ei_hbm.at[1].at[s], dst_v)
    pltpu.sync_copy(w_hbm.at[s], w_v)

    cvec = jnp.full((LANES,), c, jnp.int32)

    def _xform(i, carry):
        sl = pl.ds(i * LANES, LANES)
        v = src_v[sl]
        src_v[sl] = v + v + cvec
        return carry

    lax.fori_loop(0, EPT // LANES, _xform, 0)

    def _gather(jj, b):
        pltpu.async_copy(x_hbm.at[src_v.at[pl.ds(jj * CHUNK, CHUNK)]],
                         rows[b], sg[b])

    for b in range(NBUF - 1):
        _gather(jnp.int32(b), b)

    def _dstix(j):
        return dst_v.at[pl.ds(j * CHUNK, CHUNK)]

    def _round(q, carry):
        for b in range(NBUF):
            j = q * NBUF + b
            pltpu.make_async_copy(x_hbm.at[src_v.at[pl.ds(j * CHUNK, CHUNK)]],
                                  rows[b], sg[b]).wait()
            for g in range(CHUNK // LANES):
                w16 = w_v[pl.ds(j * CHUNK + g * LANES, LANES)]
                for i in range(LANES):
                    wspl = jnp.full((LANES,), w16[i])
                    e = g * LANES + i
                    for k in range(DH // LANES):
                        sl = pl.ds(k * LANES, LANES)
                        rows[b][e, sl] = rows[b][e, sl] * wspl
            pltpu.async_copy(rows[b], agg_sh.at[_dstix(j)], ss[b], add=True)
            pb = (b + NBUF - 1) % NBUF

            def _scat_wait():
                pltpu.make_async_copy(rows[pb],
                                      agg_sh.at[_dstix(j)], ss[pb]).wait()

            if b == 0:
                @pl.when(q > 0)
                def _scat_wait0():
                    _scat_wait()
            else:
                _scat_wait()

            @pl.when(j + NBUF - 1 < NCHUNK)
            def _refill():
                _gather(j + NBUF - 1, pb)
        return carry

    lax.fori_loop(0, NCHUNK // NBUF, _round, 0)
    pltpu.make_async_copy(rows[NBUF - 1], agg_sh.at[_dstix(0)],
                          ss[NBUF - 1]).wait()
    plsc.subcore_barrier()

    for rep in range(RPT // ZROWS):
        off = pl.multiple_of(base + rep * ZROWS, 16)
        pltpu.sync_copy(agg_sh.at[pl.ds(off, ZROWS)],
                        out_hbm.at[c, pl.ds(off, ZROWS)])

    @pl.when(s == NS - 1)
    def _write_tail():
        pltpu.sync_copy(agg_sh.at[pl.ds(NS * RPT, 16)],
                        out_hbm.at[c, pl.ds(NS * RPT, 16)])


_sc_aggregate = pl.kernel(
    _sc_body,
    out_type=jax.ShapeDtypeStruct((NC, N, DH), jnp.float32),
    mesh=plsc.VectorSubcoreMesh(core_axis_name="c", subcore_axis_name="s"),
    scratch_types=[
        pltpu.VMEM((EPT,), jnp.int32),
        pltpu.VMEM((EPT,), jnp.int32),
        pltpu.VMEM((EPT,), jnp.float32),
        pltpu.VMEM((CHUNK, DH), jnp.float32),
        pltpu.VMEM((CHUNK, DH), jnp.float32),
        pltpu.VMEM((CHUNK, DH), jnp.float32),
        pltpu.VMEM((CHUNK, DH), jnp.float32),
        pltpu.VMEM((CHUNK, DH), jnp.float32),
        pltpu.VMEM((ZROWS, DH), jnp.float32),
        pltpu.VMEM_SHARED((N, DH), jnp.float32),
        pltpu.SemaphoreType.DMA,
        pltpu.SemaphoreType.DMA,
        pltpu.SemaphoreType.DMA,
        pltpu.SemaphoreType.DMA,
        pltpu.SemaphoreType.DMA,
        pltpu.SemaphoreType.DMA,
        pltpu.SemaphoreType.DMA,
        pltpu.SemaphoreType.DMA,
        pltpu.SemaphoreType.DMA,
        pltpu.SemaphoreType.DMA,
    ],
    compiler_params=pltpu.CompilerParams(use_tc_tiling_on_sc=False),
)


NB = 2000
NBLK = N // NB


def _tc_body(agg2, Wg, bg, W1c, b1c, W2b, b2c, W3b, b3c, pwr, y, m,
             out_ref, acc):
    i = pl.program_id(0)

    @pl.when(i == 0)
    def _init():
        for k in range(len(PWS) + 1):
            acc[k] = 0.0

    emb = jnp.maximum(
        jnp.dot(agg2[0], Wg[:DH], preferred_element_type=jnp.float32)
        + jnp.dot(agg2[1], Wg[DH:], preferred_element_type=jnp.float32)
        + bg[...], 0.0)
    yv = y[...]
    mv = m[...]
    acc[len(PWS)] = acc[len(PWS)] + jnp.sum(mv)
    h = jnp.maximum(
        jnp.dot(emb, W1c[...], preferred_element_type=jnp.float32)
        + b1c[...], 0.0)
    h = jnp.maximum(
        jnp.dot(h, W2b[...], preferred_element_type=jnp.float32)
        + b2c[...], 0.0)
    z = jnp.dot(h, W3b[...], preferred_element_type=jnp.float32) + b3c[...]
    pw = pwr[...]
    l = (1.0 - yv) * z + (1.0 + (pw - 1.0) * yv) * (
        jnp.log1p(jnp.exp(-jnp.abs(z))) + jnp.maximum(-z, 0.0))
    ls = jnp.sum(l * mv, axis=0)
    for hd in range(len(PWS)):
        acc[hd] = acc[hd] + ls[hd]

    @pl.when(i == NBLK - 1)
    def _fin():
        denom = jnp.maximum(acc[len(PWS)], 1.0)
        for hd in range(len(PWS)):
            out_ref[hd] = acc[hd] / denom


def _tc_dense(agg2, Wg, bg, W1c, b1c, W2b, b2c, W3b, b3c, pwr, y, m):
    full = lambda shape: pl.BlockSpec(shape, lambda i: tuple(0 for _ in shape))
    nh = len(PWS)
    return pl.pallas_call(
        _tc_body,
        grid=(NBLK,),
        in_specs=[
            pl.BlockSpec((NC, NB, DH), lambda i: (0, i, 0)),
            full((D, D)),
            full((1, D)),
            full((D, nh * H1)),
            full((1, nh * H1)),
            full((nh * H1, nh * H2)),
            full((1, nh * H2)),
            full((nh * H2, nh)),
            full((1, nh)),
            full((1, nh)),
            pl.BlockSpec((NB, 1), lambda i: (i, 0)),
            pl.BlockSpec((NB, 1), lambda i: (i, 0)),
        ],
        out_shape=jax.ShapeDtypeStruct((nh,), jnp.float32),
        out_specs=pl.BlockSpec(memory_space=pltpu.SMEM),
        scratch_shapes=[pltpu.SMEM((nh + 1,), jnp.float32)],
        compiler_params=pltpu.CompilerParams(
            dimension_semantics=("arbitrary",)),
    )(agg2, Wg, bg, W1c, b1c, W2b, b2c, W3b, b3c, pwr, y, m)


def _block_diag(blocks):
    rs = sum(b.shape[0] for b in blocks)
    cs = sum(b.shape[1] for b in blocks)
    out = jnp.zeros((rs, cs), jnp.float32)
    r = c = 0
    for b in blocks:
        out = out.at[r:r + b.shape[0], c:c + b.shape[1]].set(b)
        r += b.shape[0]
        c += b.shape[1]
    return out


def kernel(x, edge_index, edge_weight, y, mask, W_gnn, b_gnn,
           pr_W1, pr_b1, pr_W2, pr_b2, pr_W3, pr_b3,
           aux_W1, aux_b1, aux_W2, aux_b2, aux_W3, aux_b3):
    agg2 = _sc_aggregate(edge_index.reshape(2, NS, EPT),
                         edge_weight.reshape(NS, EPT),
                         x.reshape(NC * N, DH))

    nh = len(PWS)
    W1c = jnp.moveaxis(jnp.concatenate([pr_W1[None], aux_W1]), 0, 1)
    W1c = W1c.reshape(D, nh * H1)
    b1c = jnp.concatenate([pr_b1[None], aux_b1]).reshape(1, nh * H1)
    W2b = _block_diag([pr_W2, aux_W2[0], aux_W2[1], aux_W2[2]])
    b2c = jnp.concatenate([pr_b2[None], aux_b2]).reshape(1, nh * H2)
    W3b = _block_diag([pr_W3, aux_W3[0], aux_W3[1], aux_W3[2]])
    b3c = jnp.concatenate([pr_b3[None], aux_b3]).reshape(1, nh)

    pwr = jnp.asarray(PWS, jnp.float32).reshape(1, nh)
    return _tc_dense(agg2, W_gnn, b_gnn.reshape(1, D), W1c, b1c, W2b, b2c,
                     W3b, b3c, pwr, y.reshape(N, 1),
                     mask.astype(jnp.float32).reshape(N, 1))
